# baseline (speedup 1.0000x reference)
import functools

import jax
import jax.numpy as jnp
from jax import lax
from jax.experimental import pallas as pl
from jax.experimental.pallas import tpu as pltpu
from jax.experimental.pallas import tpu_sc as plsc

_B = 1024
_NREL = 237
_NENT = 14541
_NW = 32
_BPW = _B // _NW


def _sc_body(e2ed_hbm, e2e_hbm, e2r_hbm, w_hbm, ents_hbm, tl_hbm,
             out_hbm,
             w_v, ents_v, tl_v,
             aidx, a_dup, a_rdup, cidx, c_flat, didx, d_e, d_r,
             out1_v, feats_v, sem, esem, rsem, wsem):
    wid = lax.axis_index("s") * 2 + lax.axis_index("c")
    base = wid * _BPW
    iota = lax.iota(jnp.int32, 16)
    half_s = iota >> 1
    lsb = iota & 1
    low8 = iota < 8
    s8 = iota & 7
    woff = jnp.where(low8, 64, 128)

    wcopy = pltpu.async_copy(w_hbm, w_v, wsem)
    pltpu.sync_copy(ents_hbm.at[pl.ds(base * 2, 2 * _BPW)], ents_v)
    pltpu.sync_copy(tl_hbm.at[pl.ds(base * 16, 16 * _BPW)], tl_v)

    ev = None
    for c in range(2 * _BPW):
        if c % 16 == 0:
            ev = ents_v[pl.ds(c, 16)]
        ent = ev[c % 16]
        aidx[pl.ds(c * 16, 16)] = ent * 8 + half_s

    a_waits = [pltpu.async_copy(e2ed_hbm.at[aidx.at[pl.ds(c * 128, 128)]],
                                a_dup.at[pl.ds(c * 128, 128)], sem)
               for c in range(32 * _BPW // 128)]
    for h in a_waits:
        h.wait()

    r_waits = [pltpu.async_copy(e2r_hbm.at[a_dup.at[pl.ds(c * 128, 128)]],
                                a_rdup.at[pl.ds(c * 128, 128)], sem)
               for c in range(32 * _BPW // 128)]

    def cidx_body(c, carry):
        dup = a_dup[pl.ds(c * 16, 16)]
        cidx[pl.ds(c * 16, 16)] = dup * 2 + lsb
        return carry
    lax.fori_loop(0, 2 * _BPW, cidx_body, None)
    for h in r_waits:
        h.wait()

    waits = [pltpu.async_copy(e2e_hbm.at[cidx.at[pl.ds(c * 128, 128)]],
                              c_flat.at[pl.ds(c * 128, 128)], sem)
             for c in range(32 * _BPW // 128)]
    for h in waits:
        h.wait()

    def didx_body(g, carry):
        cv = c_flat[pl.ds(g * 16, 16)]
        for h in range(8):
            ent = jnp.where(low8, cv[2 * h], cv[2 * h + 1])
            didx[pl.ds(g * 128 + h * 16, 16)] = ent * 8 + s8
        return carry
    lax.fori_loop(0, 2 * _BPW, didx_body, None)

    e_waits = [pltpu.async_copy(e2ed_hbm.at[didx.at[pl.ds(c * 128, 128)]],
                                d_e.at[pl.ds(c * 128, 128)], esem)
               for c in range(256 * _BPW // 128)]

    def b_body(j, carry):
        tlv = tl_v[pl.ds(j * 16, 16)]
        te = tlv[0]
        lab = tlv[8]
        te_vec = jnp.broadcast_to(te, (16,))

        rr = [a_rdup[pl.ds(j * 32, 16)], a_rdup[pl.ds(j * 32 + 16, 16)]]
        for k in range(16):
            rel1 = rr[k >> 3][(k & 7) * 2]
            base16 = (j * 32 + k * 2) * 8
            evv = d_e[pl.ds(base16, 16)]
            rvv = d_r[pl.ds(base16, 16)]
            addrv = jnp.where(evv == te_vec, _NREL, rvv) * 192 + woff
            accs = [w_v[pl.ds(rel1 * 192 + g * 16, 16)] for g in range(4)]
            for l in range(16):
                a0 = addrv[l]
                accs = [accs[g] + w_v[pl.ds(a0 + g * 16, 16)]
                        for g in range(4)]
            for g in range(4):
                out1_v[k, pl.ds(g * 16, 16)] = jnp.maximum(accs[g], 0.0)

        ee = [a_dup[pl.ds(j * 32, 16)], a_dup[pl.ds(j * 32 + 16, 16)]]
        sel = [jnp.where(ee[0] == te_vec, _NREL, rr[0]) * 192 + 64,
               jnp.where(ee[1] == te_vec, _NREL, rr[1]) * 192 + 128]
        msk = [jnp.where(ee[0] == te_vec, 0.0, 0.125),
               jnp.where(ee[1] == te_vec, 0.0, 0.125)]

        accs = [w_v[pl.ds(lab * 192 + g * 16, 16)] for g in range(4)]
        for k in range(16):
            a0 = sel[k >> 3][(k & 7) * 2]
            accs = [accs[g] + w_v[pl.ds(a0 + g * 16, 16)] for g in range(4)]
        for g in range(4):
            feats_v[j, pl.ds(g * 16, 16)] = jnp.maximum(accs[g], 0.0)

        for j2 in range(2):
            accs = [jnp.zeros((16,), jnp.float32) for _ in range(4)]
            for s in range(8):
                m = msk[j2][s * 2]
                accs = [accs[g] + m * out1_v[j2 * 8 + s, pl.ds(g * 16, 16)]
                        for g in range(4)]
            for g in range(4):
                feats_v[j, pl.ds(64 + j2 * 64 + g * 16, 16)] = accs[g]
        return carry

    r2_waits = []
    for c in range(256 * _BPW // 128):
        e_waits[c].wait()
        r2_waits.append(
            pltpu.async_copy(e2r_hbm.at[d_e.at[pl.ds(c * 128, 128)]],
                             d_r.at[pl.ds(c * 128, 128)], rsem))
    for h in r2_waits:
        h.wait()
    wcopy.wait()
    lax.fori_loop(0, _BPW, b_body, None)

    pltpu.sync_copy(feats_v, out_hbm.at[pl.ds(base, _BPW)])


_sc_feats = functools.partial(
    pl.kernel,
    out_type=jax.ShapeDtypeStruct((_B, 192), jnp.float32),
    mesh=plsc.VectorSubcoreMesh(core_axis_name="c", subcore_axis_name="s"),
    compiler_params=pltpu.CompilerParams(use_tc_tiling_on_sc=False),
    scratch_types=[
        pltpu.VMEM((238 * 192,), jnp.float32),
        pltpu.VMEM((2 * _BPW,), jnp.int32),
        pltpu.VMEM((16 * _BPW,), jnp.int32),
        pltpu.VMEM((32 * _BPW,), jnp.int32),
        pltpu.VMEM((32 * _BPW,), jnp.int32),
        pltpu.VMEM((32 * _BPW,), jnp.int32),
        pltpu.VMEM((32 * _BPW,), jnp.int32),
        pltpu.VMEM((32 * _BPW,), jnp.int32),
        pltpu.VMEM((256 * _BPW,), jnp.int32),
        pltpu.VMEM((256 * _BPW,), jnp.int32),
        pltpu.VMEM((256 * _BPW,), jnp.int32),
        pltpu.VMEM((16, 64), jnp.float32),
        pltpu.VMEM((_BPW, 192), jnp.float32),
        pltpu.SemaphoreType.DMA,
        pltpu.SemaphoreType.DMA,
        pltpu.SemaphoreType.DMA,
        pltpu.SemaphoreType.DMA,
    ],
)(_sc_body)


def _tc_head(f_ref, w_ref, b_ref, o_ref):
    x = jnp.dot(f_ref[...], w_ref[...], preferred_element_type=jnp.float32)
    x = x + b_ref[...]
    o_ref[...] = 1.0 / (1.0 + jnp.exp(-x))


def kernel(relation_features, W0, b0, W1, b1, labels, entity_pairs, train_edges,
           entity2edges, edge2entities, edge2relation):
    w0x = jnp.concatenate(
        [W0[:_NREL] + b0[None, :], W0[_NREL:2 * _NREL] * 0.125,
         W0[2 * _NREL:] * 0.125], axis=1)
    w0x = jnp.pad(w0x, ((0, 1), (0, 0))).reshape(-1)
    e2ed_flat = entity2edges.reshape(-1)
    e2e_flat = edge2entities.reshape(-1)
    ents = entity_pairs.reshape(-1)
    tl = jnp.concatenate(
        [jnp.broadcast_to(train_edges[:, None], (_B, 8)),
         jnp.broadcast_to(labels[:, None], (_B, 8))], axis=1).reshape(-1)

    feats = _sc_feats(e2ed_flat, e2e_flat, edge2relation, w0x, ents, tl)

    return pl.pallas_call(
        _tc_head,
        out_shape=jax.ShapeDtypeStruct((_B, _NREL), jnp.float32),
    )(feats, W1, b1.reshape(1, _NREL))


# speedup vs baseline: 2.1790x; 2.1790x over previous
import functools

import jax
import jax.numpy as jnp
from jax import lax
from jax.experimental import pallas as pl
from jax.experimental.pallas import tpu as pltpu
from jax.experimental.pallas import tpu_sc as plsc

_B = 1024
_NREL = 237
_NENT = 14541
_NEDGE1 = 272116
_NW = 32
_BPW = _B // _NW


def _sc_body(e2ed_hbm, e2e_hbm, e2r_hbm, w_hbm, ents_hbm, tl_hbm,
             out_hbm,
             w_v, ents_v, tl_v,
             aidx, a_dup, a_rdup, cidx, c_flat, didx, d_e, d_r,
             out1_v, feats_v, sem, esem, rsem, wsem):
    wid = lax.axis_index("s") * 2 + lax.axis_index("c")
    base = wid * _BPW
    iota = lax.iota(jnp.int32, 16)
    half_s = iota >> 1
    lsb = iota & 1
    low8 = iota < 8
    s8 = iota & 7
    woff = jnp.where(low8, 64, 128)
    hs_ent = half_s * _NENT
    s8_ent = s8 * _NENT
    lsb_edge = lsb * _NEDGE1

    wcopy = pltpu.async_copy(w_hbm, w_v, wsem)
    pltpu.sync_copy(ents_hbm.at[pl.ds(base * 2, 2 * _BPW)], ents_v)
    pltpu.sync_copy(tl_hbm.at[pl.ds(base * 16, 16 * _BPW)], tl_v)

    ev = None
    for c in range(2 * _BPW):
        if c % 16 == 0:
            ev = ents_v[pl.ds(c, 16)]
        ent = ev[c % 16]
        aidx[pl.ds(c * 16, 16)] = hs_ent + ent

    a_waits = [pltpu.async_copy(e2ed_hbm.at[aidx.at[pl.ds(c * 128, 128)]],
                                a_dup.at[pl.ds(c * 128, 128)], sem)
               for c in range(32 * _BPW // 128)]
    for h in a_waits:
        h.wait()

    r_waits = [pltpu.async_copy(e2r_hbm.at[a_dup.at[pl.ds(c * 128, 128)]],
                                a_rdup.at[pl.ds(c * 128, 128)], sem)
               for c in range(32 * _BPW // 128)]

    def cidx_body(c, carry):
        dup = a_dup[pl.ds(c * 16, 16)]
        cidx[pl.ds(c * 16, 16)] = dup + lsb_edge
        return carry
    lax.fori_loop(0, 2 * _BPW, cidx_body, None)
    for h in r_waits:
        h.wait()

    waits = [pltpu.async_copy(e2e_hbm.at[cidx.at[pl.ds(c * 128, 128)]],
                              c_flat.at[pl.ds(c * 128, 128)], sem)
             for c in range(32 * _BPW // 128)]
    for h in waits:
        h.wait()

    def didx_body(g, carry):
        cv = c_flat[pl.ds(g * 16, 16)]
        for h in range(8):
            ent = jnp.where(low8, cv[2 * h], cv[2 * h + 1])
            didx[pl.ds(g * 128 + h * 16, 16)] = s8_ent + ent
        return carry
    lax.fori_loop(0, 2 * _BPW, didx_body, None)

    e_waits = [pltpu.async_copy(e2ed_hbm.at[didx.at[pl.ds(c * 128, 128)]],
                                d_e.at[pl.ds(c * 128, 128)], esem)
               for c in range(256 * _BPW // 128)]

    def b_body(j, carry):
        tlv = tl_v[pl.ds(j * 16, 16)]
        te = tlv[0]
        lab = tlv[8]
        te_vec = jnp.broadcast_to(te, (16,))

        rr = [a_rdup[pl.ds(j * 32, 16)], a_rdup[pl.ds(j * 32 + 16, 16)]]
        for k in range(16):
            rel1 = rr[k >> 3][(k & 7) * 2]
            base16 = (j * 32 + k * 2) * 8
            evv = d_e[pl.ds(base16, 16)]
            rvv = d_r[pl.ds(base16, 16)]
            addrv = jnp.where(evv == te_vec, _NREL, rvv) * 192 + woff
            accs = [w_v[pl.ds(rel1 * 192 + g * 16, 16)] for g in range(4)]
            for l in range(16):
                a0 = addrv[l]
                accs = [accs[g] + w_v[pl.ds(a0 + g * 16, 16)]
                        for g in range(4)]
            for g in range(4):
                out1_v[k, pl.ds(g * 16, 16)] = jnp.maximum(accs[g], 0.0)

        ee = [a_dup[pl.ds(j * 32, 16)], a_dup[pl.ds(j * 32 + 16, 16)]]
        sel = [jnp.where(ee[0] == te_vec, _NREL, rr[0]) * 192 + 64,
               jnp.where(ee[1] == te_vec, _NREL, rr[1]) * 192 + 128]
        msk = [jnp.where(ee[0] == te_vec, 0.0, 0.125),
               jnp.where(ee[1] == te_vec, 0.0, 0.125)]

        accs = [w_v[pl.ds(lab * 192 + g * 16, 16)] for g in range(4)]
        for k in range(16):
            a0 = sel[k >> 3][(k & 7) * 2]
            accs = [accs[g] + w_v[pl.ds(a0 + g * 16, 16)] for g in range(4)]
        for g in range(4):
            feats_v[j, pl.ds(g * 16, 16)] = jnp.maximum(accs[g], 0.0)

        for j2 in range(2):
            accs = [jnp.zeros((16,), jnp.float32) for _ in range(4)]
            for s in range(8):
                m = msk[j2][s * 2]
                accs = [accs[g] + m * out1_v[j2 * 8 + s, pl.ds(g * 16, 16)]
                        for g in range(4)]
            for g in range(4):
                feats_v[j, pl.ds(64 + j2 * 64 + g * 16, 16)] = accs[g]
        return carry

    r2_waits = []
    for c in range(256 * _BPW // 128):
        e_waits[c].wait()
        r2_waits.append(
            pltpu.async_copy(e2r_hbm.at[d_e.at[pl.ds(c * 128, 128)]],
                             d_r.at[pl.ds(c * 128, 128)], rsem))
    for h in r2_waits:
        h.wait()
    wcopy.wait()
    lax.fori_loop(0, _BPW, b_body, None)

    pltpu.sync_copy(feats_v, out_hbm.at[pl.ds(base, _BPW)])


_sc_feats = functools.partial(
    pl.kernel,
    out_type=jax.ShapeDtypeStruct((_B, 192), jnp.float32),
    mesh=plsc.VectorSubcoreMesh(core_axis_name="c", subcore_axis_name="s"),
    compiler_params=pltpu.CompilerParams(use_tc_tiling_on_sc=False),
    scratch_types=[
        pltpu.VMEM((238 * 192,), jnp.float32),
        pltpu.VMEM((2 * _BPW,), jnp.int32),
        pltpu.VMEM((16 * _BPW,), jnp.int32),
        pltpu.VMEM((32 * _BPW,), jnp.int32),
        pltpu.VMEM((32 * _BPW,), jnp.int32),
        pltpu.VMEM((32 * _BPW,), jnp.int32),
        pltpu.VMEM((32 * _BPW,), jnp.int32),
        pltpu.VMEM((32 * _BPW,), jnp.int32),
        pltpu.VMEM((256 * _BPW,), jnp.int32),
        pltpu.VMEM((256 * _BPW,), jnp.int32),
        pltpu.VMEM((256 * _BPW,), jnp.int32),
        pltpu.VMEM((16, 64), jnp.float32),
        pltpu.VMEM((_BPW, 192), jnp.float32),
        pltpu.SemaphoreType.DMA,
        pltpu.SemaphoreType.DMA,
        pltpu.SemaphoreType.DMA,
        pltpu.SemaphoreType.DMA,
    ],
)(_sc_body)


def _tc_head(f_ref, w_ref, b_ref, o_ref):
    x = jnp.dot(f_ref[...], w_ref[...], preferred_element_type=jnp.float32)
    x = x + b_ref[...]
    o_ref[...] = 1.0 / (1.0 + jnp.exp(-x))


def kernel(relation_features, W0, b0, W1, b1, labels, entity_pairs, train_edges,
           entity2edges, edge2entities, edge2relation):
    w0x = jnp.concatenate(
        [W0[:_NREL] + b0[None, :], W0[_NREL:2 * _NREL] * 0.125,
         W0[2 * _NREL:] * 0.125], axis=1)
    w0x = jnp.pad(w0x, ((0, 1), (0, 0))).reshape(-1)
    e2ed_flat = entity2edges.T.reshape(-1)
    e2e_flat = edge2entities.T.reshape(-1)
    ents = entity_pairs.reshape(-1)
    tl = jnp.concatenate(
        [jnp.broadcast_to(train_edges[:, None], (_B, 8)),
         jnp.broadcast_to(labels[:, None], (_B, 8))], axis=1).reshape(-1)

    feats = _sc_feats(e2ed_flat, e2e_flat, edge2relation, w0x, ents, tl)

    return pl.pallas_call(
        _tc_head,
        out_shape=jax.ShapeDtypeStruct((_B, _NREL), jnp.float32),
    )(feats, W1, b1.reshape(1, _NREL))
